# baseline (speedup 1.0000x reference)
import functools

import jax
import jax.numpy as jnp
from jax import lax
from jax.experimental import pallas as pl
from jax.experimental.pallas import tpu as pltpu
from jax.experimental.pallas import tpu_sc as plsc

_B, _H, _W, _D = 16, 32, 32, 256
_NC, _NS, _L = 2, 16, 16


def _sc_body(col_hbm, row_hbm, out_hbm, chunk, row_buf, sem):
    i = lax.axis_index("s") * _NC + lax.axis_index("c")
    pltpu.sync_copy(col_hbm.at[pl.ds(0, _W), :], chunk.at[:, pl.ds(0, _D)])
    pltpu.sync_copy(row_hbm.at[i, :], row_buf)
    for t in range(_D // _L):
        v = row_buf[pl.ds(t * _L, _L)]
        for j in range(_W):
            chunk[j, pl.ds(_D + t * _L, _L)] = v
    handles = [pltpu.async_copy(chunk, out_hbm.at[b, i], sem)
               for b in range(_B)]
    for hnd in handles:
        hnd.wait()


def kernel(x, row_embed, col_embed):
    b = x.shape[0]
    h, w = x.shape[-2], x.shape[-1]
    d = row_embed.shape[1]
    mesh = plsc.VectorSubcoreMesh(core_axis_name="c", subcore_axis_name="s")
    run = functools.partial(
        pl.kernel,
        mesh=mesh,
        out_type=jax.ShapeDtypeStruct((b, h, w, 2 * d), jnp.float32),
        scratch_types=[
            pltpu.VMEM((w, 2 * d), jnp.float32),
            pltpu.VMEM((d,), jnp.float32),
            pltpu.SemaphoreType.DMA,
        ],
    )(_sc_body)
    out = run(col_embed, row_embed)
    return jnp.transpose(out, (0, 3, 1, 2))


# speedup vs baseline: 2.2423x; 2.2423x over previous
import jax
import jax.numpy as jnp
from jax.experimental import pallas as pl
from jax.experimental.pallas import tpu as pltpu

_REP = 2


def _pos_kernel(col_ref, row_ref, out_ref, scratch, sems):
    r, h, w, d2 = scratch.shape
    d = d2 // 2
    b = out_ref.shape[0]
    scratch[:, :, :, :d] = jnp.broadcast_to(
        col_ref[...][None, None, :, :], (r, h, w, d))
    scratch[:, :, :, d:] = jnp.broadcast_to(
        row_ref[...][None, :, None, :], (r, h, w, d))
    n = b // r
    copies = [pltpu.make_async_copy(
        scratch, out_ref.at[pl.ds(i * r, r)], sems.at[i % sems.shape[0]])
        for i in range(n)]
    for c in copies:
        c.start()
    for c in copies:
        c.wait()


def kernel(x, row_embed, col_embed):
    b = x.shape[0]
    h, w = x.shape[-2], x.shape[-1]
    d = row_embed.shape[1]
    out = pl.pallas_call(
        _pos_kernel,
        in_specs=[
            pl.BlockSpec(memory_space=pltpu.MemorySpace.VMEM),
            pl.BlockSpec(memory_space=pltpu.MemorySpace.VMEM),
        ],
        out_specs=pl.BlockSpec(memory_space=pl.ANY),
        out_shape=jax.ShapeDtypeStruct((b, h, w, 2 * d), jnp.float32),
        scratch_shapes=[
            pltpu.VMEM((_REP, h, w, 2 * d), jnp.float32),
            pltpu.SemaphoreType.DMA((8,)),
        ],
    )(col_embed[:w], row_embed[:h])
    return jnp.transpose(out, (0, 3, 1, 2))
